# Initial kernel scaffold; baseline (speedup 1.0000x reference)
import jax
import jax.numpy as jnp
from jax.experimental import pallas as pl


def kernel(page_hash_bucket, offset_bucket, cache_line_bucket, alignment_bucket, stride_bucket, reuse_dist_bucket, locality_cluster, entropy_bucket, flags, page_table, offset_table, cache_table, align_table, stride_table, reuse_table, locality_table, entropy_table, flags_W, flags_b, gate_W, gate_b, trans_W, trans_b, ln_g, ln_b):
    raise NotImplementedError("write your pallas kernel here")


# speedup vs baseline: 4.6600x; 4.6600x over previous
import functools

import jax
import jax.numpy as jnp
from jax import lax
from jax.experimental import pallas as pl
from jax.experimental.pallas import tpu as pltpu
from jax.experimental.pallas import tpu_sc as plsc

B, L = 4096, 50
N = B * L
D_OUT = 128
DIMS = (8, 4, 4, 3, 6, 5, 6, 3)
E_TOT = sum(DIMS)
NT = len(DIMS)
PD = 8

NC, NS = 2, 16
NW = NC * NS
TOK_W = N // NW
CHUNK = 128
NCH = TOK_W // CHUNK


def _sc_gather(idx_hbm, tbl_hbm, out_hbm, idx_v, rows_v, gsems, wsems):
    wid = lax.axis_index("s") * NC + lax.axis_index("c")
    base = wid * TOK_W

    for k in range(NT):
        pltpu.sync_copy(idx_hbm[k].at[wid], idx_v[k])

    def fire(j, slot):
        for k in range(NT):
            pltpu.async_copy(tbl_hbm[k].at[idx_v[k].at[j]],
                             rows_v[k].at[slot], gsems[slot])

    def drain_writeback(j, slot):
        for k in range(NT):
            pltpu.make_async_copy(tbl_hbm[k].at[idx_v[k].at[j]],
                                  rows_v[k].at[slot], gsems[slot]).wait()
        start = base + j * CHUNK
        for k in range(NT):
            pltpu.async_copy(rows_v[k].at[slot],
                             out_hbm[k].at[pl.ds(start, CHUNK)], wsems[slot])

    def wb_wait(j, slot):
        start = base + j * CHUNK
        for k in range(NT):
            pltpu.make_async_copy(rows_v[k].at[slot],
                                  out_hbm[k].at[pl.ds(start, CHUNK)],
                                  wsems[slot]).wait()

    fire(0, 0)

    def body(j2, carry):
        j = j2 * 2
        @pl.when(j2 > 0)
        def _():
            wb_wait(j - 1, 1)
        fire(j + 1, 1)
        drain_writeback(j, 0)
        @pl.when(j2 < NCH // 2 - 1)
        def _():
            wb_wait(j, 0)
            fire(j + 2, 0)
        drain_writeback(j + 1, 1)
        return carry

    lax.fori_loop(0, NCH // 2, body, 0, unroll=False)
    wb_wait(NCH - 2, 0)
    wb_wait(NCH - 1, 1)


@functools.lru_cache(maxsize=1)
def _make_sc_call():
    mesh = plsc.VectorSubcoreMesh(core_axis_name="c", subcore_axis_name="s")
    out_type = [jax.ShapeDtypeStruct((N, PD), jnp.float32) for _ in DIMS]
    scratch = ([pltpu.VMEM((NCH, CHUNK), jnp.int32) for _ in DIMS]
               + [pltpu.VMEM((2, CHUNK, PD), jnp.float32) for _ in DIMS]
               + [pltpu.SemaphoreType.DMA, pltpu.SemaphoreType.DMA,
                  pltpu.SemaphoreType.DMA, pltpu.SemaphoreType.DMA])

    @functools.partial(
        pl.kernel, mesh=mesh, out_type=out_type, scratch_types=scratch,
        compiler_params=pltpu.CompilerParams(use_tc_tiling_on_sc=False),
        name="addr_enc_gather")
    def call(*refs):
        idx_hbm = refs[:NT]
        tbl_hbm = refs[NT:2 * NT]
        out_hbm = refs[2 * NT:3 * NT]
        idx_v = refs[3 * NT:4 * NT]
        rows_v = refs[4 * NT:5 * NT]
        gsems = refs[5 * NT:5 * NT + 2]
        wsems = refs[5 * NT + 2:5 * NT + 4]
        _sc_gather(list(idx_hbm), list(tbl_hbm), list(out_hbm),
                   list(idx_v), list(rows_v), list(gsems), list(wsems))

    return call


TN = 1024


def _tc_fuse_kernel(flags_ref, Wf_ref, bias_ref, W64_ref, lg_ref, lb_ref,
                    *e_refs, o_ref):
    e = jnp.concatenate([r[...] for r in e_refs], axis=1)
    acc = (jnp.dot(e, W64_ref[...], preferred_element_type=jnp.float32)
           + jnp.dot(flags_ref[...], Wf_ref[...],
                     preferred_element_type=jnp.float32)
           + bias_ref[...])
    gate = jax.nn.sigmoid(acc[:, :D_OUT] * 1.2)
    z = gate * acc[:, D_OUT:]
    mu = jnp.mean(z, axis=-1, keepdims=True)
    zc = z - mu
    var = jnp.mean(zc * zc, axis=-1, keepdims=True)
    o_ref[...] = zc * jax.lax.rsqrt(var + 1e-5) * lg_ref[...] + lb_ref[...]


def _tc_fuse(e_list, flags2d, Wf, bias, W64, lg, lb):
    grid = (N // TN,)
    row_spec = lambda d: pl.BlockSpec((TN, d), lambda i: (i, 0))
    w_spec = lambda r, c: pl.BlockSpec((r, c), lambda i: (0, 0))
    kernel_fn = lambda *args: _tc_fuse_kernel(*args[:-1], o_ref=args[-1])
    return pl.pallas_call(
        kernel_fn,
        grid=grid,
        in_specs=[row_spec(5), w_spec(5, 2 * D_OUT), w_spec(1, 2 * D_OUT),
                  w_spec(NT * PD, 2 * D_OUT),
                  w_spec(1, D_OUT), w_spec(1, D_OUT)]
                 + [row_spec(PD) for _ in DIMS],
        out_specs=pl.BlockSpec((TN, D_OUT), lambda i: (i, 0)),
        out_shape=jax.ShapeDtypeStruct((N, D_OUT), jnp.float32),
    )(flags2d, Wf, bias, W64, lg.reshape(1, D_OUT), lb.reshape(1, D_OUT),
      *e_list)


def kernel(page_hash_bucket, offset_bucket, cache_line_bucket, alignment_bucket,
           stride_bucket, reuse_dist_bucket, locality_cluster, entropy_bucket,
           flags, page_table, offset_table, cache_table, align_table,
           stride_table, reuse_table, locality_table, entropy_table,
           flags_W, flags_b, gate_W, gate_b, trans_W, trans_b, ln_g, ln_b):
    idxs = [page_hash_bucket, offset_bucket, cache_line_bucket,
            alignment_bucket, stride_bucket, reuse_dist_bucket,
            locality_cluster, entropy_bucket]
    tbls = [page_table, offset_table, cache_table, align_table, stride_table,
            reuse_table, locality_table, entropy_table]
    idxs = [i.astype(jnp.int32).reshape(NW, NCH, CHUNK) for i in idxs]
    tbls = [t if t.shape[1] == PD else
            jnp.pad(t, ((0, 0), (0, PD - t.shape[1]))) for t in tbls]
    e_list = _make_sc_call()(*idxs, *tbls)

    gt = jnp.concatenate([gate_W, trans_W], axis=1)
    pieces = []
    off = 0
    for d in DIMS:
        pieces.append(gt[off:off + d])
        if d < PD:
            pieces.append(jnp.zeros((PD - d, 2 * D_OUT), jnp.float32))
        off += d
    W64 = jnp.concatenate(pieces, axis=0)
    Wf = flags_W @ gt[E_TOT:]
    bias = (flags_b @ gt[E_TOT:]
            + jnp.concatenate([gate_b, trans_b])).reshape(1, 2 * D_OUT)

    out = _tc_fuse(e_list, flags.reshape(N, 5), Wf, bias, W64, ln_g, ln_b)
    return out.reshape(B, L, D_OUT)
